# Initial kernel scaffold; baseline (speedup 1.0000x reference)
import jax
import jax.numpy as jnp
from jax.experimental import pallas as pl


def kernel(x, edge_index, W_self0, W_neigh0, b0, W_self1, W_neigh1, b1, W_self2, W_neigh2, b2):
    raise NotImplementedError("write your pallas kernel here")


# speedup vs baseline: 3.5838x; 3.5838x over previous
import functools

import jax
import jax.numpy as jnp
from jax import lax
from jax.experimental import pallas as pl
from jax.experimental.pallas import tpu as pltpu
from jax.experimental.pallas import tpu_sc as plsc

N = 10000
NP = 10240
E = 320000
D = 128
DH = D // 2
NC, NS = 2, 16
EPT = E // NS
CHUNK = 80
NCHUNK = EPT // CHUNK
RPT = NP // NS
ZROWS = 128


def _agg_body(with_deg, *args):
  if with_deg:
    (h_hbm, src_hbm, dst_hbm, dst16_hbm, oh_hbm,
     out_hbm, deg_hbm, srcv, dstv, dst16v, rows, zbuf, ohv, zbuf16,
     dacc2, acc) = args
  else:
    (h_hbm, src_hbm, dst_hbm, out_hbm, srcv, dstv, rows, zbuf, acc) = args
  cid = lax.axis_index("c")
  sid = lax.axis_index("s")

  pltpu.sync_copy(src_hbm.at[sid], srcv)
  pltpu.sync_copy(dst_hbm.at[sid], dstv)
  if with_deg:
    pltpu.sync_copy(dst16_hbm.at[sid], dst16v)

  zero16 = jnp.zeros((16,), jnp.float32)

  def zb(i, _):
    zbuf[i // (DH // 16), pl.ds((i % (DH // 16)) * 16, 16)] = zero16
    return 0

  lax.fori_loop(0, ZROWS * (DH // 16), zb, 0)
  for j in range(RPT // ZROWS):
    pltpu.sync_copy(zbuf, acc.at[pl.ds(sid * RPT + j * ZROWS, ZROWS)])

  if with_deg:
    def zd(i, _):
      zbuf16[i, pl.ds(0, 16)] = zero16
      return 0

    lax.fori_loop(0, NP // 16 // NS, zd, 0)
    pltpu.sync_copy(zbuf16, dacc2.at[pl.ds(sid * (NP // 16 // NS),
                                           NP // 16 // NS)])

  plsc.subcore_barrier()

  def step(j, _):
    pltpu.sync_copy(h_hbm.at[cid].at[srcv.at[j]], rows)
    pltpu.sync_copy(rows, acc.at[dstv.at[j]], add=True)
    if with_deg:
      @pl.when(cid == 0)
      def _():
        pltpu.sync_copy(oh_hbm.at[sid * NCHUNK + j], ohv)
        pltpu.sync_copy(ohv, dacc2.at[dst16v.at[j]], add=True)
    return 0

  lax.fori_loop(0, NCHUNK, step, 0)

  plsc.subcore_barrier()

  pltpu.sync_copy(acc.at[pl.ds(sid * RPT, RPT)],
                  out_hbm.at[pl.ds(cid * NP + sid * RPT, RPT)])
  if with_deg:
    @pl.when(jnp.logical_and(cid == 0, sid == 0))
    def _():
      pltpu.sync_copy(dacc2, deg_hbm)


def _make_agg(with_deg):
  mesh = plsc.VectorSubcoreMesh(core_axis_name="c", subcore_axis_name="s")
  out_type = [jax.ShapeDtypeStruct((NC * NP, DH), jnp.float32)]
  scratch = [
      pltpu.VMEM((NCHUNK, CHUNK), jnp.int32),
      pltpu.VMEM((NCHUNK, CHUNK), jnp.int32),
      pltpu.VMEM((CHUNK, DH), jnp.float32),
      pltpu.VMEM((ZROWS, DH), jnp.float32),
  ]
  if with_deg:
    out_type.append(jax.ShapeDtypeStruct((NP // 16, 16), jnp.float32))
    scratch.insert(2, pltpu.VMEM((NCHUNK, CHUNK), jnp.int32))
    scratch.append(pltpu.VMEM((CHUNK, 16), jnp.float32))
    scratch.append(pltpu.VMEM((NP // 16 // NS, 16), jnp.float32))
    scratch.append(pltpu.VMEM_SHARED((NP // 16, 16), jnp.float32))
  scratch.append(pltpu.VMEM_SHARED((NP, DH), jnp.float32))
  return pl.kernel(
      functools.partial(_agg_body, with_deg),
      out_type=tuple(out_type) if with_deg else out_type[0],
      mesh=mesh,
      scratch_types=scratch,
      compiler_params=pltpu.CompilerParams(use_tc_tiling_on_sc=False),
  )


def _dense(h01, agg, degp, w_self, w_neigh, b, last):
  BR = 512
  G = NP // BR

  def body(h_ref, a_ref, d_ref, ws_ref, wn_ref, b_ref, o_ref):
    h = jnp.concatenate([h_ref[0], h_ref[1]], axis=1)
    hn = jnp.concatenate([a_ref[0], a_ref[1]], axis=1)
    deg = d_ref[...]
    recip = 1.0 / jnp.maximum(deg, 1.0)
    acc = jnp.dot(h, ws_ref[...], preferred_element_type=jnp.float32)
    acc = acc + jnp.dot(hn * recip, wn_ref[...],
                        preferred_element_type=jnp.float32)
    acc = acc + b_ref[...]
    if last:
      o_ref[...] = acc
    else:
      acc = jnp.maximum(acc, 0.0)
      o_ref[0] = acc[:, :DH]
      o_ref[1] = acc[:, DH:]

  if last:
    out_shape = jax.ShapeDtypeStruct((NP, D), jnp.float32)
    out_specs = pl.BlockSpec((BR, D), lambda i: (i, 0))
  else:
    out_shape = jax.ShapeDtypeStruct((NC, NP, DH), jnp.float32)
    out_specs = pl.BlockSpec((NC, BR, DH), lambda i: (0, i, 0))
  return pl.pallas_call(
      body,
      grid=(G,),
      in_specs=[
          pl.BlockSpec((NC, BR, DH), lambda i: (0, i, 0)),
          pl.BlockSpec((NC, BR, DH), lambda i: (0, i, 0)),
          pl.BlockSpec((BR, 1), lambda i: (i, 0)),
          pl.BlockSpec((D, D), lambda i: (0, 0)),
          pl.BlockSpec((D, D), lambda i: (0, 0)),
          pl.BlockSpec((1, D), lambda i: (0, 0)),
      ],
      out_specs=out_specs,
      out_shape=out_shape,
  )(h01, agg, degp, w_self, w_neigh, b.reshape(1, D))


_agg_deg = _make_agg(True)
_agg = _make_agg(False)


@jax.jit
def kernel(x, edge_index, W_self0, W_neigh0, b0, W_self1, W_neigh1, b1,
           W_self2, W_neigh2, b2):
  src = edge_index[0].astype(jnp.int32).reshape(NS, NCHUNK, CHUNK)
  dstf = edge_index[1].astype(jnp.int32)
  dst = dstf.reshape(NS, NCHUNK, CHUNK)
  dst16 = (dstf // 16).reshape(NS, NCHUNK, CHUNK)
  oh = jax.nn.one_hot(dstf % 16, 16, dtype=jnp.float32)
  oh = oh.reshape(NS * NCHUNK, CHUNK, 16)
  x_pad = jnp.zeros((NP, D), jnp.float32).at[:N].set(x)
  x01 = x_pad.reshape(NP, NC, DH).transpose(1, 0, 2)
  agg, degp = _agg_deg(x01, src, dst, dst16, oh)
  degp = degp.reshape(NP, 1)
  h01 = _dense(x01, agg.reshape(NC, NP, DH), degp, W_self0, W_neigh0, b0,
               last=False)
  agg = _agg(h01, src, dst)
  h01 = _dense(h01, agg.reshape(NC, NP, DH), degp, W_self1, W_neigh1, b1,
               last=False)
  agg = _agg(h01, src, dst)
  return _dense(h01, agg.reshape(NC, NP, DH), degp, W_self2, W_neigh2, b2,
                last=True)[:N]


# speedup vs baseline: 6.6507x; 1.8558x over previous
import jax
import jax.numpy as jnp
from jax import lax
from jax.experimental import pallas as pl
from jax.experimental.pallas import tpu as pltpu
from jax.experimental.pallas import tpu_sc as plsc

N = 10000
NP = 10240
E = 320000
D = 128
DH = D // 2
NC, NS = 2, 16
NW = NC * NS
CHUNK = 100
NCHUNK = E // NS // CHUNK
NCHUNKD = E // NW // CHUNK
NBUF = 4
NCYC = NCHUNK // NBUF
RPT = NP // NS
ZROWS = 128
ND = NP // 16


def _agg_body(h_hbm, src_hbm, dst_hbm, out_hbm, *rest):
  srcv, dstv = rest[:2]
  rows = rest[2:2 + NBUF]
  zbuf = rest[2 + NBUF]
  gsem = rest[3 + NBUF:3 + 2 * NBUF]
  ssem = rest[3 + 2 * NBUF:3 + 3 * NBUF]
  acc = rest[3 + 3 * NBUF]
  cid = lax.axis_index("c")
  sid = lax.axis_index("s")

  pltpu.sync_copy(src_hbm.at[sid], srcv)
  pltpu.sync_copy(dst_hbm.at[sid], dstv)

  for b in range(NBUF):
    pltpu.async_copy(h_hbm.at[cid].at[srcv.at[b]], rows[b], gsem[b])

  zero16 = jnp.zeros((16,), jnp.float32)

  def zb(i, _):
    zbuf[i // (DH // 16), pl.ds((i % (DH // 16)) * 16, 16)] = zero16
    return 0

  lax.fori_loop(0, ZROWS * (DH // 16), zb, 0)
  for j in range(RPT // ZROWS):
    pltpu.sync_copy(zbuf, acc.at[pl.ds(sid * RPT + j * ZROWS, ZROWS)])

  plsc.subcore_barrier()

  def wait_g(b):
    pltpu.make_async_copy(h_hbm.at[cid].at[srcv.at[0]], rows[b],
                          gsem[b]).wait()

  def wait_s(b):
    pltpu.make_async_copy(rows[b], acc.at[dstv.at[0]], ssem[b]).wait()

  def cycle(g, last):
    for b in range(NBUF):
      j = NBUF * g + b
      wait_g(b)
      pltpu.async_copy(rows[b], acc.at[dstv.at[j]], ssem[b], add=True)
    if last:
      for b in range(NBUF):
        wait_s(b)
    else:
      for b in range(NBUF):
        jn = NBUF * (g + 1) + b
        wait_s(b)
        pltpu.async_copy(h_hbm.at[cid].at[srcv.at[jn]], rows[b], gsem[b])

  def cyc_body(g, _):
    cycle(g, False)
    return 0

  lax.fori_loop(0, NCYC - 1, cyc_body, 0)
  cycle(NCYC - 1, True)

  plsc.subcore_barrier()

  pltpu.sync_copy(acc.at[pl.ds(sid * RPT, RPT)],
                  out_hbm.at[pl.ds(cid * NP + sid * RPT, RPT)])


def _deg_body(dst16_hbm, oh_hbm, deg_hbm, dst16v, oh0, oh1, zbuf16,
              og0, og1, ds0, ds1, dacc2):
  ohv, ogsem, dsem = [oh0, oh1], [og0, og1], [ds0, ds1]
  cid = lax.axis_index("c")
  sid = lax.axis_index("s")
  wid = cid * NS + sid

  pltpu.sync_copy(dst16_hbm.at[wid], dst16v)
  for kb in range(2):
    pltpu.async_copy(oh_hbm.at[wid * NCHUNKD + kb], ohv[kb], ogsem[kb])

  zero16 = jnp.zeros((16,), jnp.float32)

  def zd(i, _):
    zbuf16[i, pl.ds(0, 16)] = zero16
    return 0

  lax.fori_loop(0, ND // NS, zd, 0)
  pltpu.sync_copy(zbuf16, dacc2.at[pl.ds(sid * (ND // NS), ND // NS)])

  plsc.subcore_barrier()

  def wait_og(kb):
    pltpu.make_async_copy(oh_hbm.at[0], ohv[kb], ogsem[kb]).wait()

  def wait_ds(kb):
    pltpu.make_async_copy(ohv[kb], dacc2.at[dst16v.at[0]], dsem[kb]).wait()

  def dcycle(g, last):
    for kb in range(2):
      j = 2 * g + kb
      wait_og(kb)
      pltpu.async_copy(ohv[kb], dacc2.at[dst16v.at[j]], dsem[kb], add=True)
    if last:
      for kb in range(2):
        wait_ds(kb)
    else:
      for kb in range(2):
        jn = 2 * (g + 1) + kb
        wait_ds(kb)
        pltpu.async_copy(oh_hbm.at[wid * NCHUNKD + jn], ohv[kb], ogsem[kb])

  def dcyc_body(g, _):
    dcycle(g, False)
    return 0

  lax.fori_loop(0, NCHUNKD // 2 - 1, dcyc_body, 0)
  dcycle(NCHUNKD // 2 - 1, True)

  plsc.subcore_barrier()

  @pl.when(sid == 0)
  def _():
    pltpu.sync_copy(dacc2, deg_hbm.at[cid])


_agg = pl.kernel(
    _agg_body,
    out_type=jax.ShapeDtypeStruct((NC * NP, DH), jnp.float32),
    mesh=plsc.VectorSubcoreMesh(core_axis_name="c", subcore_axis_name="s"),
    scratch_types=(
        [pltpu.VMEM((NCHUNK, CHUNK), jnp.int32)] * 2
        + [pltpu.VMEM((CHUNK, DH), jnp.float32)] * NBUF
        + [pltpu.VMEM((ZROWS, DH), jnp.float32)]
        + [pltpu.SemaphoreType.DMA] * (2 * NBUF)
        + [pltpu.VMEM_SHARED((NP, DH), jnp.float32)]
    ),
    compiler_params=pltpu.CompilerParams(use_tc_tiling_on_sc=False),
)

_deg = pl.kernel(
    _deg_body,
    out_type=jax.ShapeDtypeStruct((NC, ND, 16), jnp.float32),
    mesh=plsc.VectorSubcoreMesh(core_axis_name="c", subcore_axis_name="s"),
    scratch_types=(
        [pltpu.VMEM((NCHUNKD, CHUNK), jnp.int32)]
        + [pltpu.VMEM((CHUNK, 16), jnp.float32)] * 2
        + [pltpu.VMEM((ND // NS, 16), jnp.float32)]
        + [pltpu.SemaphoreType.DMA] * 4
        + [pltpu.VMEM_SHARED((ND, 16), jnp.float32)]
    ),
    compiler_params=pltpu.CompilerParams(use_tc_tiling_on_sc=False),
)


def _dense(h01, agg, degp, w_self, w_neigh, b, flag):
  BR = 512
  G = NP // BR

  def body(h_ref, a_ref, d_ref, ws_ref, wn_ref, b_ref, f_ref, o_ref):
    h = jnp.concatenate([h_ref[0], h_ref[1]], axis=1)
    hn = jnp.concatenate([a_ref[0], a_ref[1]], axis=1)
    deg = d_ref[0] + d_ref[1]
    recip = 1.0 / jnp.maximum(deg, 1.0)
    acc = jnp.dot(h, ws_ref[...], preferred_element_type=jnp.float32)
    acc = acc + jnp.dot(hn * recip, wn_ref[...],
                        preferred_element_type=jnp.float32)
    acc = acc + b_ref[...]
    acc = jnp.where(f_ref[0, 0] > 0.0, jnp.maximum(acc, 0.0), acc)
    o_ref[0] = acc[:, :DH]
    o_ref[1] = acc[:, DH:]

  return pl.pallas_call(
      body,
      grid=(G,),
      in_specs=[
          pl.BlockSpec((NC, BR, DH), lambda i: (0, i, 0)),
          pl.BlockSpec((NC, BR, DH), lambda i: (0, i, 0)),
          pl.BlockSpec((NC, BR, 1), lambda i: (0, i, 0)),
          pl.BlockSpec((D, D), lambda i: (0, 0)),
          pl.BlockSpec((D, D), lambda i: (0, 0)),
          pl.BlockSpec((1, D), lambda i: (0, 0)),
          pl.BlockSpec((1, 1), lambda i: (0, 0)),
      ],
      out_specs=pl.BlockSpec((NC, BR, DH), lambda i: (0, i, 0)),
      out_shape=jax.ShapeDtypeStruct((NC, NP, DH), jnp.float32),
  )(h01, agg, degp, w_self, w_neigh, b.reshape(1, D), flag.reshape(1, 1))


@jax.jit
def kernel(x, edge_index, W_self0, W_neigh0, b0, W_self1, W_neigh1, b1,
           W_self2, W_neigh2, b2):
  src = edge_index[0].astype(jnp.int32).reshape(NS, NCHUNK, CHUNK)
  dstf = edge_index[1].astype(jnp.int32)
  dst = dstf.reshape(NS, NCHUNK, CHUNK)
  dst16 = (dstf // 16).reshape(NW, NCHUNKD, CHUNK)
  oh = jax.nn.one_hot(dstf % 16, 16, dtype=jnp.float32)
  oh = oh.reshape(NW * NCHUNKD, CHUNK, 16)
  x_pad = jnp.zeros((NP, D), jnp.float32).at[:N].set(x)
  x01 = x_pad.reshape(NP, NC, DH).transpose(1, 0, 2)

  degp = _deg(dst16, oh).reshape(NC, NP, 1)

  ws = jnp.stack([W_self0, W_self1, W_self2])
  wn = jnp.stack([W_neigh0, W_neigh1, W_neigh2])
  bs = jnp.stack([b0, b1, b2])
  flags = jnp.array([1.0, 1.0, 0.0], jnp.float32)

  n_iter = lax.optimization_barrier(jnp.int32(3))

  def cond(c):
    return c[0] < n_iter

  def layer(c):
    i, h = c
    agg = _agg(h, src, dst).reshape(NC, NP, DH)
    h = _dense(h, agg, degp,
               lax.dynamic_index_in_dim(ws, i, keepdims=False),
               lax.dynamic_index_in_dim(wn, i, keepdims=False),
               lax.dynamic_index_in_dim(bs, i, keepdims=False),
               lax.dynamic_index_in_dim(flags, i, keepdims=False))
    return (i + 1, h)

  _, h = lax.while_loop(cond, layer, (jnp.int32(0), x01))
  return jnp.concatenate([h[0], h[1]], axis=1)[:N]


# speedup vs baseline: 7.0868x; 1.0656x over previous
import jax
import jax.numpy as jnp
from jax import lax
from jax.experimental import pallas as pl
from jax.experimental.pallas import tpu as pltpu
from jax.experimental.pallas import tpu_sc as plsc

N = 10000
NP = 10240
E = 320000
D = 128
DH = D // 2
NC, NS = 2, 16
NW = NC * NS
CHUNK = 125
NCHUNK = E // NS // CHUNK
NCHUNKD = E // NW // CHUNK
NBUF = 4
NCYC = NCHUNK // NBUF
RPT = NP // NS
ZROWS = 128
ND = NP // 16


def _agg_body(h_hbm, src_hbm, dst_hbm, out_hbm, *rest):
  srcv, dstv = rest[:2]
  rows = rest[2:2 + NBUF]
  zbuf = rest[2 + NBUF]
  gsem = rest[3 + NBUF:3 + 2 * NBUF]
  ssem = rest[3 + 2 * NBUF:3 + 3 * NBUF]
  acc = rest[3 + 3 * NBUF]
  cid = lax.axis_index("c")
  sid = lax.axis_index("s")

  pltpu.sync_copy(src_hbm.at[sid], srcv)
  pltpu.sync_copy(dst_hbm.at[sid], dstv)

  for b in range(NBUF):
    pltpu.async_copy(h_hbm.at[cid].at[srcv.at[b]], rows[b], gsem[b])

  zero16 = jnp.zeros((16,), jnp.float32)

  def zb(i, _):
    zbuf[i // (DH // 16), pl.ds((i % (DH // 16)) * 16, 16)] = zero16
    return 0

  lax.fori_loop(0, ZROWS * (DH // 16), zb, 0)
  for j in range(RPT // ZROWS):
    pltpu.sync_copy(zbuf, acc.at[pl.ds(sid * RPT + j * ZROWS, ZROWS)])

  plsc.subcore_barrier()

  def wait_g(b):
    pltpu.make_async_copy(h_hbm.at[cid].at[srcv.at[0]], rows[b],
                          gsem[b]).wait()

  def wait_s(b):
    pltpu.make_async_copy(rows[b], acc.at[dstv.at[0]], ssem[b]).wait()

  def cycle(g, _):
    for b in range(NBUF):
      j = NBUF * g + b
      wait_g(b)
      pltpu.async_copy(rows[b], acc.at[dstv.at[j]], ssem[b], add=True)
    for b in range(NBUF):
      jn = NBUF * (g + 1) + b
      wait_s(b)

      @pl.when(jn < NCHUNK)
      def _():
        pltpu.async_copy(h_hbm.at[cid].at[srcv.at[jn]], rows[b], gsem[b])
    return 0

  lax.fori_loop(0, NCYC, cycle, 0)

  plsc.subcore_barrier()

  pltpu.sync_copy(acc.at[pl.ds(sid * RPT, RPT)],
                  out_hbm.at[pl.ds(cid * NP + sid * RPT, RPT)])


def _deg_body(dst16_hbm, oh_hbm, deg_hbm, dst16v, oh0, oh1, zbuf16,
              og0, og1, ds0, ds1, dacc2):
  ohv, ogsem, dsem = [oh0, oh1], [og0, og1], [ds0, ds1]
  cid = lax.axis_index("c")
  sid = lax.axis_index("s")
  wid = cid * NS + sid

  pltpu.sync_copy(dst16_hbm.at[wid], dst16v)
  for kb in range(2):
    pltpu.async_copy(oh_hbm.at[wid * NCHUNKD + kb], ohv[kb], ogsem[kb])

  zero16 = jnp.zeros((16,), jnp.float32)

  def zd(i, _):
    zbuf16[i, pl.ds(0, 16)] = zero16
    return 0

  lax.fori_loop(0, ND // NS, zd, 0)
  pltpu.sync_copy(zbuf16, dacc2.at[pl.ds(sid * (ND // NS), ND // NS)])

  plsc.subcore_barrier()

  def wait_og(kb):
    pltpu.make_async_copy(oh_hbm.at[0], ohv[kb], ogsem[kb]).wait()

  def wait_ds(kb):
    pltpu.make_async_copy(ohv[kb], dacc2.at[dst16v.at[0]], dsem[kb]).wait()

  def dcycle(g, _):
    for kb in range(2):
      j = 2 * g + kb
      wait_og(kb)
      pltpu.async_copy(ohv[kb], dacc2.at[dst16v.at[j]], dsem[kb], add=True)
    for kb in range(2):
      jn = 2 * (g + 1) + kb
      wait_ds(kb)

      @pl.when(jn < NCHUNKD)
      def _():
        pltpu.async_copy(oh_hbm.at[wid * NCHUNKD + jn], ohv[kb], ogsem[kb])
    return 0

  lax.fori_loop(0, NCHUNKD // 2, dcycle, 0)

  plsc.subcore_barrier()

  @pl.when(sid == 0)
  def _():
    pltpu.sync_copy(dacc2, deg_hbm.at[cid])


_agg = pl.kernel(
    _agg_body,
    out_type=jax.ShapeDtypeStruct((NC * NP, DH), jnp.float32),
    mesh=plsc.VectorSubcoreMesh(core_axis_name="c", subcore_axis_name="s"),
    scratch_types=(
        [pltpu.VMEM((NCHUNK, CHUNK), jnp.int32)] * 2
        + [pltpu.VMEM((CHUNK, DH), jnp.float32)] * NBUF
        + [pltpu.VMEM((ZROWS, DH), jnp.float32)]
        + [pltpu.SemaphoreType.DMA] * (2 * NBUF)
        + [pltpu.VMEM_SHARED((NP, DH), jnp.float32)]
    ),
    compiler_params=pltpu.CompilerParams(use_tc_tiling_on_sc=False),
)

_deg = pl.kernel(
    _deg_body,
    out_type=jax.ShapeDtypeStruct((NC, ND, 16), jnp.float32),
    mesh=plsc.VectorSubcoreMesh(core_axis_name="c", subcore_axis_name="s"),
    scratch_types=(
        [pltpu.VMEM((NCHUNKD, CHUNK), jnp.int32)]
        + [pltpu.VMEM((CHUNK, 16), jnp.float32)] * 2
        + [pltpu.VMEM((ND // NS, 16), jnp.float32)]
        + [pltpu.SemaphoreType.DMA] * 4
        + [pltpu.VMEM_SHARED((ND, 16), jnp.float32)]
    ),
    compiler_params=pltpu.CompilerParams(use_tc_tiling_on_sc=False),
)


def _dense(h01, agg, degp, w_self, w_neigh, b, flag):
  BR = 512
  G = NP // BR

  def body(h_ref, a_ref, d_ref, ws_ref, wn_ref, b_ref, f_ref, o_ref):
    h = jnp.concatenate([h_ref[0], h_ref[1]], axis=1)
    hn = jnp.concatenate([a_ref[0], a_ref[1]], axis=1)
    deg = d_ref[0] + d_ref[1]
    recip = 1.0 / jnp.maximum(deg, 1.0)
    acc = jnp.dot(h, ws_ref[...], preferred_element_type=jnp.float32)
    acc = acc + jnp.dot(hn * recip, wn_ref[...],
                        preferred_element_type=jnp.float32)
    acc = acc + b_ref[...]
    acc = jnp.where(f_ref[0, 0] > 0.0, jnp.maximum(acc, 0.0), acc)
    o_ref[0] = acc[:, :DH]
    o_ref[1] = acc[:, DH:]

  return pl.pallas_call(
      body,
      grid=(G,),
      in_specs=[
          pl.BlockSpec((NC, BR, DH), lambda i: (0, i, 0)),
          pl.BlockSpec((NC, BR, DH), lambda i: (0, i, 0)),
          pl.BlockSpec((NC, BR, 1), lambda i: (0, i, 0)),
          pl.BlockSpec((D, D), lambda i: (0, 0)),
          pl.BlockSpec((D, D), lambda i: (0, 0)),
          pl.BlockSpec((1, D), lambda i: (0, 0)),
          pl.BlockSpec((1, 1), lambda i: (0, 0)),
      ],
      out_specs=pl.BlockSpec((NC, BR, DH), lambda i: (0, i, 0)),
      out_shape=jax.ShapeDtypeStruct((NC, NP, DH), jnp.float32),
  )(h01, agg, degp, w_self, w_neigh, b.reshape(1, D), flag.reshape(1, 1))


@jax.jit
def kernel(x, edge_index, W_self0, W_neigh0, b0, W_self1, W_neigh1, b1,
           W_self2, W_neigh2, b2):
  src = edge_index[0].astype(jnp.int32).reshape(NS, NCHUNK, CHUNK)
  dstf = edge_index[1].astype(jnp.int32)
  dst = dstf.reshape(NS, NCHUNK, CHUNK)
  dst16 = (dstf // 16).reshape(NW, NCHUNKD, CHUNK)
  oh = jax.nn.one_hot(dstf % 16, 16, dtype=jnp.float32)
  oh = oh.reshape(NW * NCHUNKD, CHUNK, 16)
  x_pad = jnp.zeros((NP, D), jnp.float32).at[:N].set(x)
  x01 = x_pad.reshape(NP, NC, DH).transpose(1, 0, 2)

  degp = _deg(dst16, oh).reshape(NC, NP, 1)

  ws = jnp.stack([W_self0, W_self1, W_self2])
  wn = jnp.stack([W_neigh0, W_neigh1, W_neigh2])
  bs = jnp.stack([b0, b1, b2])
  flags = jnp.array([1.0, 1.0, 0.0], jnp.float32)

  n_iter = lax.optimization_barrier(jnp.int32(3))

  def cond(c):
    return c[0] < n_iter

  def layer(c):
    i, h = c
    agg = _agg(h, src, dst).reshape(NC, NP, DH)
    h = _dense(h, agg, degp,
               lax.dynamic_index_in_dim(ws, i, keepdims=False),
               lax.dynamic_index_in_dim(wn, i, keepdims=False),
               lax.dynamic_index_in_dim(bs, i, keepdims=False),
               lax.dynamic_index_in_dim(flags, i, keepdims=False))
    return (i + 1, h)

  _, h = lax.while_loop(cond, layer, (jnp.int32(0), x01))
  return jnp.concatenate([h[0], h[1]], axis=1)[:N]


# speedup vs baseline: 7.3036x; 1.0306x over previous
import jax
import jax.numpy as jnp
from jax import lax
from jax.experimental import pallas as pl
from jax.experimental.pallas import tpu as pltpu
from jax.experimental.pallas import tpu_sc as plsc

N = 10000
NP = 10240
E = 320000
D = 128
DH = D // 2
NC, NS = 2, 16
NW = NC * NS
CHUNK = 125
NCHUNK = E // NS // CHUNK
NCHUNKD = E // NW // CHUNK
NBUF = 4
NCYC = NCHUNK // NBUF
RPT = NP // NS
ZROWS = 128
ND = NP // 16


def _agg_body(h_hbm, src_hbm, dst_hbm, out_hbm, *rest):
  srcv, dstv = rest[:2]
  rows = rest[2:2 + NBUF]
  zbuf = rest[2 + NBUF]
  gsem = rest[3 + NBUF:3 + 2 * NBUF]
  ssem = rest[3 + 2 * NBUF:3 + 3 * NBUF]
  acc = rest[3 + 3 * NBUF]
  cid = lax.axis_index("c")
  sid = lax.axis_index("s")

  pltpu.sync_copy(src_hbm.at[sid], srcv)
  pltpu.sync_copy(dst_hbm.at[sid], dstv)

  for b in range(NBUF):
    pltpu.async_copy(h_hbm.at[cid].at[srcv.at[b]], rows[b], gsem[b])

  zero16 = jnp.zeros((16,), jnp.float32)

  def zb(i, _):
    zbuf[i // (DH // 16), pl.ds((i % (DH // 16)) * 16, 16)] = zero16
    return 0

  lax.fori_loop(0, ZROWS * (DH // 16), zb, 0)
  for j in range(RPT // ZROWS):
    pltpu.sync_copy(zbuf, acc.at[pl.ds(sid * RPT + j * ZROWS, ZROWS)])

  plsc.subcore_barrier()

  def wait_g(b):
    pltpu.make_async_copy(h_hbm.at[cid].at[srcv.at[0]], rows[b],
                          gsem[b]).wait()

  def wait_s(b):
    pltpu.make_async_copy(rows[b], acc.at[dstv.at[0]], ssem[b]).wait()

  def cycle(g, _):
    for b in range(NBUF):
      j = NBUF * g + b
      wait_g(b)
      pltpu.async_copy(rows[b], acc.at[dstv.at[j]], ssem[b], add=True)
    for b in range(NBUF):
      jn = NBUF * (g + 1) + b
      wait_s(b)

      @pl.when(jn < NCHUNK)
      def _():
        pltpu.async_copy(h_hbm.at[cid].at[srcv.at[jn]], rows[b], gsem[b])
    return 0

  lax.fori_loop(0, NCYC, cycle, 0)

  plsc.subcore_barrier()

  pltpu.sync_copy(acc.at[pl.ds(sid * RPT, RPT)],
                  out_hbm.at[pl.ds(cid * NP + sid * RPT, RPT)])


def _deg_body(dst16_hbm, oh_hbm, deg_hbm, dst16v, oh0, oh1, zbuf16,
              og0, og1, ds0, ds1, dacc2):
  ohv, ogsem, dsem = [oh0, oh1], [og0, og1], [ds0, ds1]
  cid = lax.axis_index("c")
  sid = lax.axis_index("s")
  wid = cid * NS + sid

  pltpu.sync_copy(dst16_hbm.at[wid], dst16v)
  for kb in range(2):
    pltpu.async_copy(oh_hbm.at[wid * NCHUNKD + kb], ohv[kb], ogsem[kb])

  zero16 = jnp.zeros((16,), jnp.float32)

  def zd(i, _):
    zbuf16[i, pl.ds(0, 16)] = zero16
    return 0

  lax.fori_loop(0, ND // NS, zd, 0)
  pltpu.sync_copy(zbuf16, dacc2.at[pl.ds(sid * (ND // NS), ND // NS)])

  plsc.subcore_barrier()

  def wait_og(kb):
    pltpu.make_async_copy(oh_hbm.at[0], ohv[kb], ogsem[kb]).wait()

  def wait_ds(kb):
    pltpu.make_async_copy(ohv[kb], dacc2.at[dst16v.at[0]], dsem[kb]).wait()

  def dcycle(g, _):
    for kb in range(2):
      j = 2 * g + kb
      wait_og(kb)
      pltpu.async_copy(ohv[kb], dacc2.at[dst16v.at[j]], dsem[kb], add=True)
    for kb in range(2):
      jn = 2 * (g + 1) + kb
      wait_ds(kb)

      @pl.when(jn < NCHUNKD)
      def _():
        pltpu.async_copy(oh_hbm.at[wid * NCHUNKD + jn], ohv[kb], ogsem[kb])
    return 0

  lax.fori_loop(0, NCHUNKD // 2, dcycle, 0)

  plsc.subcore_barrier()

  @pl.when(sid == 0)
  def _():
    pltpu.sync_copy(dacc2, deg_hbm.at[cid])


_agg = pl.kernel(
    _agg_body,
    out_type=jax.ShapeDtypeStruct((NC * NP, DH), jnp.float32),
    mesh=plsc.VectorSubcoreMesh(core_axis_name="c", subcore_axis_name="s"),
    scratch_types=(
        [pltpu.VMEM((NCHUNK, CHUNK), jnp.int32)] * 2
        + [pltpu.VMEM((CHUNK, DH), jnp.float32)] * NBUF
        + [pltpu.VMEM((ZROWS, DH), jnp.float32)]
        + [pltpu.SemaphoreType.DMA] * (2 * NBUF)
        + [pltpu.VMEM_SHARED((NP, DH), jnp.float32)]
    ),
    compiler_params=pltpu.CompilerParams(use_tc_tiling_on_sc=False),
)

_deg = pl.kernel(
    _deg_body,
    out_type=jax.ShapeDtypeStruct((NC, ND, 16), jnp.float32),
    mesh=plsc.VectorSubcoreMesh(core_axis_name="c", subcore_axis_name="s"),
    scratch_types=(
        [pltpu.VMEM((NCHUNKD, CHUNK), jnp.int32)]
        + [pltpu.VMEM((CHUNK, 16), jnp.float32)] * 2
        + [pltpu.VMEM((ND // NS, 16), jnp.float32)]
        + [pltpu.SemaphoreType.DMA] * 4
        + [pltpu.VMEM_SHARED((ND, 16), jnp.float32)]
    ),
    compiler_params=pltpu.CompilerParams(use_tc_tiling_on_sc=False),
)


def _selfterm(h01, w_self, b):
  BR = 512
  G = NP // BR

  def body(h_ref, ws_ref, b_ref, o_ref):
    h = jnp.concatenate([h_ref[0], h_ref[1]], axis=1)
    o_ref[...] = jnp.dot(h, ws_ref[...],
                         preferred_element_type=jnp.float32) + b_ref[...]

  return pl.pallas_call(
      body,
      grid=(G,),
      in_specs=[
          pl.BlockSpec((NC, BR, DH), lambda i: (0, i, 0)),
          pl.BlockSpec((D, D), lambda i: (0, 0)),
          pl.BlockSpec((1, D), lambda i: (0, 0)),
      ],
      out_specs=pl.BlockSpec((BR, D), lambda i: (i, 0)),
      out_shape=jax.ShapeDtypeStruct((NP, D), jnp.float32),
  )(h01, w_self, b.reshape(1, D))


def _combine(st, agg, degp, w_neigh, flag):
  BR = 512
  G = NP // BR

  def body(st_ref, a_ref, d_ref, wn_ref, f_ref, o_ref):
    hn = jnp.concatenate([a_ref[0], a_ref[1]], axis=1)
    deg = d_ref[0] + d_ref[1]
    recip = 1.0 / jnp.maximum(deg, 1.0)
    acc = st_ref[...] + jnp.dot(hn * recip, wn_ref[...],
                                preferred_element_type=jnp.float32)
    acc = jnp.where(f_ref[0, 0] > 0.0, jnp.maximum(acc, 0.0), acc)
    o_ref[0] = acc[:, :DH]
    o_ref[1] = acc[:, DH:]

  return pl.pallas_call(
      body,
      grid=(G,),
      in_specs=[
          pl.BlockSpec((BR, D), lambda i: (i, 0)),
          pl.BlockSpec((NC, BR, DH), lambda i: (0, i, 0)),
          pl.BlockSpec((NC, BR, 1), lambda i: (0, i, 0)),
          pl.BlockSpec((D, D), lambda i: (0, 0)),
          pl.BlockSpec((1, 1), lambda i: (0, 0)),
      ],
      out_specs=pl.BlockSpec((NC, BR, DH), lambda i: (0, i, 0)),
      out_shape=jax.ShapeDtypeStruct((NC, NP, DH), jnp.float32),
  )(st, agg, degp, w_neigh, flag.reshape(1, 1))


@jax.jit
def kernel(x, edge_index, W_self0, W_neigh0, b0, W_self1, W_neigh1, b1,
           W_self2, W_neigh2, b2):
  src = edge_index[0].astype(jnp.int32).reshape(NS, NCHUNK, CHUNK)
  dstf = edge_index[1].astype(jnp.int32)
  dst = dstf.reshape(NS, NCHUNK, CHUNK)
  dst16 = (dstf // 16).reshape(NW, NCHUNKD, CHUNK)
  oh = jax.nn.one_hot(dstf % 16, 16, dtype=jnp.float32)
  oh = oh.reshape(NW * NCHUNKD, CHUNK, 16)
  x_pad = jnp.zeros((NP, D), jnp.float32).at[:N].set(x)
  x01 = x_pad.reshape(NP, NC, DH).transpose(1, 0, 2)

  degp = _deg(dst16, oh).reshape(NC, NP, 1)

  ws = jnp.stack([W_self0, W_self1, W_self2])
  wn = jnp.stack([W_neigh0, W_neigh1, W_neigh2])
  bs = jnp.stack([b0, b1, b2])
  flags = jnp.array([1.0, 1.0, 0.0], jnp.float32)

  n_iter = lax.optimization_barrier(jnp.int32(3))

  def cond(c):
    return c[0] < n_iter

  def layer(c):
    i, h = c
    agg = _agg(h, src, dst).reshape(NC, NP, DH)
    st = _selfterm(h, lax.dynamic_index_in_dim(ws, i, keepdims=False),
                   lax.dynamic_index_in_dim(bs, i, keepdims=False))
    h = _combine(st, agg, degp,
                 lax.dynamic_index_in_dim(wn, i, keepdims=False),
                 lax.dynamic_index_in_dim(flags, i, keepdims=False))
    return (i + 1, h)

  _, h = lax.while_loop(cond, layer, (jnp.int32(0), x01))
  return jnp.concatenate([h[0], h[1]], axis=1)[:N]


# speedup vs baseline: 7.3771x; 1.0101x over previous
import jax
import jax.numpy as jnp
from jax import lax
from jax.experimental import pallas as pl
from jax.experimental.pallas import tpu as pltpu
from jax.experimental.pallas import tpu_sc as plsc

N = 10000
NP = 10240
E = 320000
D = 128
DH = D // 2
NC, NS = 2, 16
NW = NC * NS
CHUNK = 125
NCHUNK = E // NS // CHUNK
NCHUNKD = E // NW // CHUNK
NBUF = 5
NCYC = NCHUNK // NBUF
RPT = NP // NS
ZROWS = 128
ND = NP // 16


def _agg_body(h_hbm, src_hbm, dst_hbm, out_hbm, *rest):
  srcv, dstv = rest[:2]
  rows = rest[2:2 + NBUF]
  zbuf = rest[2 + NBUF]
  gsem = rest[3 + NBUF:3 + 2 * NBUF]
  ssem = rest[3 + 2 * NBUF:3 + 3 * NBUF]
  acc = rest[3 + 3 * NBUF]
  cid = lax.axis_index("c")
  sid = lax.axis_index("s")

  pltpu.sync_copy(src_hbm.at[sid], srcv)
  pltpu.sync_copy(dst_hbm.at[sid], dstv)

  for b in range(NBUF):
    pltpu.async_copy(h_hbm.at[cid].at[srcv.at[b]], rows[b], gsem[b])

  zero16 = jnp.zeros((16,), jnp.float32)

  def zb(i, _):
    zbuf[i // (DH // 16), pl.ds((i % (DH // 16)) * 16, 16)] = zero16
    return 0

  lax.fori_loop(0, ZROWS * (DH // 16), zb, 0)
  for j in range(RPT // ZROWS):
    pltpu.sync_copy(zbuf, acc.at[pl.ds(sid * RPT + j * ZROWS, ZROWS)])

  plsc.subcore_barrier()

  def wait_g(b):
    pltpu.make_async_copy(h_hbm.at[cid].at[srcv.at[0]], rows[b],
                          gsem[b]).wait()

  def wait_s(b):
    pltpu.make_async_copy(rows[b], acc.at[dstv.at[0]], ssem[b]).wait()

  def cycle(g, _):
    for b in range(NBUF):
      j = NBUF * g + b
      wait_g(b)
      pltpu.async_copy(rows[b], acc.at[dstv.at[j]], ssem[b], add=True)
    for b in range(NBUF):
      jn = NBUF * (g + 1) + b
      wait_s(b)

      @pl.when(jn < NCHUNK)
      def _():
        pltpu.async_copy(h_hbm.at[cid].at[srcv.at[jn]], rows[b], gsem[b])
    return 0

  lax.fori_loop(0, NCYC, cycle, 0)

  plsc.subcore_barrier()

  pltpu.sync_copy(acc.at[pl.ds(sid * RPT, RPT)],
                  out_hbm.at[pl.ds(cid * NP + sid * RPT, RPT)])


def _deg_body(dst16_hbm, oh_hbm, deg_hbm, dst16v, oh0, oh1, zbuf16,
              og0, og1, ds0, ds1, dacc2):
  ohv, ogsem, dsem = [oh0, oh1], [og0, og1], [ds0, ds1]
  cid = lax.axis_index("c")
  sid = lax.axis_index("s")
  wid = cid * NS + sid

  pltpu.sync_copy(dst16_hbm.at[wid], dst16v)
  for kb in range(2):
    pltpu.async_copy(oh_hbm.at[wid * NCHUNKD + kb], ohv[kb], ogsem[kb])

  zero16 = jnp.zeros((16,), jnp.float32)

  def zd(i, _):
    zbuf16[i, pl.ds(0, 16)] = zero16
    return 0

  lax.fori_loop(0, ND // NS, zd, 0)
  pltpu.sync_copy(zbuf16, dacc2.at[pl.ds(sid * (ND // NS), ND // NS)])

  plsc.subcore_barrier()

  def wait_og(kb):
    pltpu.make_async_copy(oh_hbm.at[0], ohv[kb], ogsem[kb]).wait()

  def wait_ds(kb):
    pltpu.make_async_copy(ohv[kb], dacc2.at[dst16v.at[0]], dsem[kb]).wait()

  def dcycle(g, _):
    for kb in range(2):
      j = 2 * g + kb
      wait_og(kb)
      pltpu.async_copy(ohv[kb], dacc2.at[dst16v.at[j]], dsem[kb], add=True)
    for kb in range(2):
      jn = 2 * (g + 1) + kb
      wait_ds(kb)

      @pl.when(jn < NCHUNKD)
      def _():
        pltpu.async_copy(oh_hbm.at[wid * NCHUNKD + jn], ohv[kb], ogsem[kb])
    return 0

  lax.fori_loop(0, NCHUNKD // 2, dcycle, 0)

  plsc.subcore_barrier()

  @pl.when(sid == 0)
  def _():
    pltpu.sync_copy(dacc2, deg_hbm.at[cid])


_agg = pl.kernel(
    _agg_body,
    out_type=jax.ShapeDtypeStruct((NC * NP, DH), jnp.float32),
    mesh=plsc.VectorSubcoreMesh(core_axis_name="c", subcore_axis_name="s"),
    scratch_types=(
        [pltpu.VMEM((NCHUNK, CHUNK), jnp.int32)] * 2
        + [pltpu.VMEM((CHUNK, DH), jnp.float32)] * NBUF
        + [pltpu.VMEM((ZROWS, DH), jnp.float32)]
        + [pltpu.SemaphoreType.DMA] * (2 * NBUF)
        + [pltpu.VMEM_SHARED((NP, DH), jnp.float32)]
    ),
    compiler_params=pltpu.CompilerParams(use_tc_tiling_on_sc=False),
)

_deg = pl.kernel(
    _deg_body,
    out_type=jax.ShapeDtypeStruct((NC, ND, 16), jnp.float32),
    mesh=plsc.VectorSubcoreMesh(core_axis_name="c", subcore_axis_name="s"),
    scratch_types=(
        [pltpu.VMEM((NCHUNKD, CHUNK), jnp.int32)]
        + [pltpu.VMEM((CHUNK, 16), jnp.float32)] * 2
        + [pltpu.VMEM((ND // NS, 16), jnp.float32)]
        + [pltpu.SemaphoreType.DMA] * 4
        + [pltpu.VMEM_SHARED((ND, 16), jnp.float32)]
    ),
    compiler_params=pltpu.CompilerParams(use_tc_tiling_on_sc=False),
)


def _selfterm(h01, w_self, b):
  BR = 512
  G = NP // BR

  def body(h_ref, ws_ref, b_ref, o_ref):
    h = jnp.concatenate([h_ref[0], h_ref[1]], axis=1)
    o_ref[...] = jnp.dot(h, ws_ref[...],
                         preferred_element_type=jnp.float32) + b_ref[...]

  return pl.pallas_call(
      body,
      grid=(G,),
      in_specs=[
          pl.BlockSpec((NC, BR, DH), lambda i: (0, i, 0)),
          pl.BlockSpec((D, D), lambda i: (0, 0)),
          pl.BlockSpec((1, D), lambda i: (0, 0)),
      ],
      out_specs=pl.BlockSpec((BR, D), lambda i: (i, 0)),
      out_shape=jax.ShapeDtypeStruct((NP, D), jnp.float32),
  )(h01, w_self, b.reshape(1, D))


def _combine(st, agg, degp, w_neigh, flag):
  BR = 512
  G = NP // BR

  def body(st_ref, a_ref, d_ref, wn_ref, f_ref, o_ref):
    hn = jnp.concatenate([a_ref[0], a_ref[1]], axis=1)
    deg = d_ref[0] + d_ref[1]
    recip = 1.0 / jnp.maximum(deg, 1.0)
    acc = st_ref[...] + jnp.dot(hn * recip, wn_ref[...],
                                preferred_element_type=jnp.float32)
    acc = jnp.where(f_ref[0, 0] > 0.0, jnp.maximum(acc, 0.0), acc)
    o_ref[0] = acc[:, :DH]
    o_ref[1] = acc[:, DH:]

  return pl.pallas_call(
      body,
      grid=(G,),
      in_specs=[
          pl.BlockSpec((BR, D), lambda i: (i, 0)),
          pl.BlockSpec((NC, BR, DH), lambda i: (0, i, 0)),
          pl.BlockSpec((NC, BR, 1), lambda i: (0, i, 0)),
          pl.BlockSpec((D, D), lambda i: (0, 0)),
          pl.BlockSpec((1, 1), lambda i: (0, 0)),
      ],
      out_specs=pl.BlockSpec((NC, BR, DH), lambda i: (0, i, 0)),
      out_shape=jax.ShapeDtypeStruct((NC, NP, DH), jnp.float32),
  )(st, agg, degp, w_neigh, flag.reshape(1, 1))


@jax.jit
def kernel(x, edge_index, W_self0, W_neigh0, b0, W_self1, W_neigh1, b1,
           W_self2, W_neigh2, b2):
  src = edge_index[0].astype(jnp.int32).reshape(NS, NCHUNK, CHUNK)
  dstf = edge_index[1].astype(jnp.int32)
  dst = dstf.reshape(NS, NCHUNK, CHUNK)
  dst16 = (dstf // 16).reshape(NW, NCHUNKD, CHUNK)
  oh = jax.nn.one_hot(dstf % 16, 16, dtype=jnp.float32)
  oh = oh.reshape(NW * NCHUNKD, CHUNK, 16)
  x_pad = jnp.zeros((NP, D), jnp.float32).at[:N].set(x)
  x01 = x_pad.reshape(NP, NC, DH).transpose(1, 0, 2)

  degp = _deg(dst16, oh).reshape(NC, NP, 1)

  ws = jnp.stack([W_self0, W_self1, W_self2])
  wn = jnp.stack([W_neigh0, W_neigh1, W_neigh2])
  bs = jnp.stack([b0, b1, b2])
  flags = jnp.array([1.0, 1.0, 0.0], jnp.float32)

  n_iter = lax.optimization_barrier(jnp.int32(3))

  def cond(c):
    return c[0] < n_iter

  def layer(c):
    i, h = c
    agg = _agg(h, src, dst).reshape(NC, NP, DH)
    st = _selfterm(h, lax.dynamic_index_in_dim(ws, i, keepdims=False),
                   lax.dynamic_index_in_dim(bs, i, keepdims=False))
    h = _combine(st, agg, degp,
                 lax.dynamic_index_in_dim(wn, i, keepdims=False),
                 lax.dynamic_index_in_dim(flags, i, keepdims=False))
    return (i + 1, h)

  _, h = lax.while_loop(cond, layer, (jnp.int32(0), x01))
  return jnp.concatenate([h[0], h[1]], axis=1)[:N]
